# Initial kernel scaffold; baseline (speedup 1.0000x reference)
import jax
import jax.numpy as jnp
from jax.experimental import pallas as pl


def kernel(x_t, probs, embedding_weight, omega_s, omega_a, omega_b):
    raise NotImplementedError("write your pallas kernel here")


# speedup vs baseline: 306.5841x; 306.5841x over previous
import functools

import jax
import jax.numpy as jnp
from jax import lax
from jax.experimental import pallas as pl
from jax.experimental.pallas import tpu as pltpu
from jax.experimental.pallas import tpu_sc as plsc

B, S, VOCAB, HIDDEN = 16, 8, 100000, 128
MASK_ID = 103
K = 5
ROWS = B * S
NWORKERS = 32
RPW = ROWS // NWORKERS
L = 16
VEC = VOCAB // L
LN2 = 0.6931471805599453
SQRT2 = 1.4142135623730951


def _vlog(x):
    bits = lax.bitcast_convert_type(x, jnp.int32)
    e = lax.shift_right_arithmetic(bits, 23) - 127
    m = lax.bitcast_convert_type(
        (bits & jnp.int32(0x007FFFFF)) | jnp.int32(0x3F800000),
        jnp.float32)
    big = m > SQRT2
    m = jnp.where(big, m * 0.5, m)
    ef = (e + big.astype(jnp.int32)).astype(jnp.float32)
    s = (m - 1.0) / (m + 1.0)
    s2 = s * s
    p = 1.0 + s2 * (1.0 / 3.0 + s2 * (0.2 + s2 * (1.0 / 7.0)))
    return ef * LN2 + 2.0 * s * p


@functools.partial(
    pl.kernel,
    mesh=plsc.VectorSubcoreMesh(core_axis_name="c", subcore_axis_name="s"),
    out_type=jax.ShapeDtypeStruct((ROWS, HIDDEN), jnp.float32),
    compiler_params=pltpu.CompilerParams(needs_layout_passes=False),
    scratch_types=[
        pltpu.VMEM((ROWS + L,), jnp.int32),
        pltpu.VMEM((L,), jnp.float32),
        pltpu.VMEM((L, HIDDEN), jnp.float32),
        pltpu.VMEM((L, HIDDEN), jnp.float32),
        pltpu.VMEM((VOCAB,), jnp.float32),
        pltpu.VMEM((HIDDEN,), jnp.float32),
        pltpu.SemaphoreType.DMA,
    ],
)
def _sc_soft_mask(xt_hbm, probs_hbm, emb_hbm, om_hbm, out_hbm,
                  xt_v, om_v, ebuf, fbuf, row_v, orow_v, sem):
    wid = lax.axis_index("c") * 16 + lax.axis_index("s")
    base = wid * RPW
    pltpu.sync_copy(xt_hbm, xt_v.at[pl.ds(0, ROWS)])
    pltpu.sync_copy(om_hbm, om_v)
    lanes = lax.iota(jnp.int32, L)
    xvec = xt_v[pl.ds(base, L)]
    xs = [xvec[j] for j in range(RPW)]
    idxv = jnp.full((L,), MASK_ID, jnp.int32)
    for j in range(RPW):
        idxv = jnp.where(lanes == j, xs[j], idxv)
    pltpu.async_copy(emb_hbm.at[idxv], ebuf, sem).wait()
    omv = om_v[...]
    ros = omv[0]
    roa = omv[1]
    rob = omv[2]
    neg = jnp.full((L,), -jnp.inf, jnp.float32)
    zf = jnp.zeros((L,), jnp.float32)
    zi = jnp.zeros((L,), jnp.int32)

    for j in range(RPW):
        r = base + j
        masked = xs[j] == MASK_ID

        @pl.when(jnp.logical_not(masked))
        def _():
            pltpu.sync_copy(ebuf.at[j], out_hbm.at[r])

        @pl.when(masked)
        def _():
            pltpu.sync_copy(probs_hbm.at[r], row_v)

            def pass1(i, carry):
                acc, best, bidx = carry
                v = row_v[pl.ds(i * L, L)]
                acc = acc + v * _vlog(v + 1e-10)
                m = v > best
                best = jnp.where(m, v, best)
                bidx = jnp.where(m, lanes + i * L, bidx)
                return acc, best, bidx

            def passk(i, carry):
                best, bidx = carry
                v = row_v[pl.ds(i * L, L)]
                m = v > best
                best = jnp.where(m, v, best)
                bidx = jnp.where(m, lanes + i * L, bidx)
                return best, bidx

            acc, best, bidx = lax.fori_loop(0, VEC, pass1, (zf, neg, zi))
            gvals, gidxs = [], []
            for k in range(K):
                if k > 0:
                    best, bidx = lax.fori_loop(0, VEC, passk, (neg, zi))
                gm = jnp.max(best)
                gi = jnp.min(jnp.where(best == gm, bidx, jnp.int32(0x7FFFFFFF)))
                gvals.append(gm)
                gidxs.append(gi)
                if k < K - 1:
                    plsc.store_scatter(row_v, [jnp.full((L,), gi, jnp.int32)],
                                       neg, mask=lanes == 0)

            gidxv = jnp.full((L,), MASK_ID, jnp.int32)
            for k in range(K):
                gidxv = jnp.where(lanes == k, gidxs[k], gidxv)
            pltpu.async_copy(emb_hbm.at[gidxv], fbuf, sem).wait()

            sumwv = jnp.full((L,), gvals[0] + gvals[1] + gvals[2]
                             + gvals[3] + gvals[4], jnp.float32)
            invv = jnp.full((L,), 1.0, jnp.float32) / sumwv
            wv = [jnp.full((L,), gvals[k], jnp.float32) * invv for k in range(K)]
            sacc = jnp.sum(acc)
            inner = roa * (sacc - rob)
            innerv = jnp.full((L,), inner, jnp.float32)
            rosv = jnp.full((L,), ros, jnp.float32)
            lamv = rosv / (1.0 + jnp.exp(-innerv))
            for h in range(HIDDEN // L):
                sl = pl.ds(h * L, L)
                fb = wv[0] * fbuf[0, sl]
                fb = fb + wv[1] * fbuf[1, sl]
                fb = fb + wv[2] * fbuf[2, sl]
                fb = fb + wv[3] * fbuf[3, sl]
                fb = fb + wv[4] * fbuf[4, sl]
                orow_v[sl] = lamv * fb + (1.0 - lamv) * ebuf[RPW, sl]
            pltpu.sync_copy(orow_v, out_hbm.at[r])


def kernel(x_t, probs, embedding_weight, omega_s, omega_a, omega_b):
    xt = x_t.reshape(ROWS).astype(jnp.int32)
    p2 = probs.reshape(ROWS, VOCAB)
    ros = jax.nn.sigmoid(omega_s).astype(jnp.float32)
    roa = jnp.logaddexp(omega_a, 0.0).astype(jnp.float32)
    rob = omega_b.astype(jnp.float32)
    om = jnp.zeros((L,), jnp.float32).at[0].set(ros).at[1].set(roa).at[2].set(rob)
    out = _sc_soft_mask(xt, p2, embedding_weight, om)
    return out.reshape(B, S, HIDDEN)
